# baseline (speedup 1.0000x reference)
import numpy as np
import jax
import jax.numpy as jnp
from jax import lax
from jax.experimental import pallas as pl
from jax.experimental.pallas import tpu as pltpu

_B, _F, _D = 4096, 100, 64
_C = _F * _D
_COLS_PER_BLOCK = 256

_U32 = np.uint32


def _np_threefry2x32(k1, k2, x0, x1):
    u = lambda v: np.array(v, dtype=_U32)
    rotl = lambda x, d: (x << u(d)) | (x >> u(32 - d))
    ks = [u(k1), u(k2), u(k1) ^ u(k2) ^ u(0x1BD11BDA)]
    rot = [(13, 15, 26, 6), (17, 29, 16, 24)]
    x0, x1 = u(x0) + ks[0], u(x1) + ks[1]
    for i in range(5):
        for r in rot[i % 2]:
            x0 = x0 + x1
            x1 = rotl(x1, r) ^ x0
        x0 = x0 + ks[(i + 1) % 3]
        x1 = x1 + ks[(i + 2) % 3] + u(i + 1)
    return x0, x1


with np.errstate(over="ignore"):
    _KM = _np_threefry2x32(0, 42, 0, 0)
    _KN = _np_threefry2x32(0, 42, 0, 1)

_LO = np.nextafter(np.float32(-1.0), np.float32(0.0), dtype=np.float32)
_HILO = np.float32(np.float32(1.0) - _LO)
_SCALE = np.float32(np.float32(np.sqrt(2.0)) * np.float32(0.05))
_P_CENTER = [np.float32(c) * _SCALE for c in (
    2.81022636e-08, 3.43273939e-07, -3.5233877e-06, -4.39150654e-06,
    0.00021858087, -0.00125372503, -0.00417768164, 0.246640727, 1.50140941)]
_P_TAIL = [np.float32(c) * _SCALE for c in (
    -0.000200214257, 0.000100950558, 0.00134934322, -0.00367342844,
    0.00573950773, -0.0076224613, 0.00943887047, 1.00167406, 2.83297682)]

_EXPAND = np.kron(np.eye(_F, dtype=np.float32),
                  np.ones((_D, 1), dtype=np.float32))


def _tf_rounds(k0, k1, x1):
    ks = [_U32(k0), _U32(k1), _U32(k0) ^ _U32(k1) ^ _U32(0x1BD11BDA)]
    rot = [(13, 15, 26, 6), (17, 29, 16, 24)]
    x0 = jnp.full_like(x1, ks[0])
    x1 = x1 + ks[1]
    for i in range(5):
        for r in rot[i % 2]:
            x0 = x0 + x1
            x1 = ((x1 << _U32(r)) | (x1 >> _U32(32 - r))) ^ x0
        x0 = x0 + ks[(i + 1) % 3]
        x1 = x1 + (ks[(i + 2) % 3] + _U32(i + 1))
    return x0, x1


def _bits(key, flat_idx_u32):
    o0, o1 = _tf_rounds(key[0], key[1], flat_idx_u32)
    return o0 ^ o1


_CHUNK = 1600


def _kern(x_ref, m_ref, e_ref, o_ref, es_ref):
    cb = x_ref.shape[1]
    col0 = pl.program_id(0) * cb

    f_i = lax.broadcasted_iota(jnp.int32, (_F, cb), 0)
    b_i = lax.broadcasted_iota(jnp.int32, (_F, cb), 1)
    midx = ((col0 + b_i) * _F + f_i).astype(jnp.uint32)
    mbits = _bits(_KM, midx)
    aug = jnp.where((mbits >> _U32(31)) == _U32(0),
                    jnp.float32(1.0), jnp.float32(0.0)) * m_ref[...]
    es_ref[...] = lax.dot_general(e_ref[...], aug, (((1,), (0,)), ((), ())),
                                  preferred_element_type=jnp.float32)

    r_j = lax.broadcasted_iota(jnp.int32, (_CHUNK, cb), 0)
    b_j = lax.broadcasted_iota(jnp.int32, (_CHUNK, cb), 1)
    base = (col0 + b_j) * _C + r_j

    def body(ch, carry):
        del carry
        sl = pl.ds(ch * _CHUNK, _CHUNK)
        nidx = (base + ch * _CHUNK).astype(jnp.uint32)
        nbits = _bits(_KN, nidx)
        fb = (nbits >> _U32(9)) | _U32(0x3F800000)
        f01 = lax.bitcast_convert_type(fb, jnp.float32) - jnp.float32(1.0)
        u = f01 * _HILO + _LO
        w = -jnp.log(jnp.float32(1.0) - u * u)
        tail = w >= jnp.float32(5.0)
        ww = jnp.where(tail, jnp.sqrt(w) - jnp.float32(3.0),
                       w - jnp.float32(2.5))
        p = jnp.where(tail, _P_TAIL[0], _P_CENTER[0])
        for cc, ct in zip(_P_CENTER[1:], _P_TAIL[1:]):
            p = p * ww + jnp.where(tail, ct, cc)
        pert = p * u
        o_ref[sl, :] = x_ref[sl, :] + pert * es_ref[sl, :]
        return 0

    for ch in range(_C // _CHUNK):
        body(ch, 0)


def kernel(input_features, attention_mask):
    xT = jnp.transpose(input_features, (1, 2, 0)).reshape(_C, _B)
    attnT = attention_mask.astype(jnp.float32).T
    nblk = _B // _COLS_PER_BLOCK
    outT = pl.pallas_call(
        _kern,
        grid=(nblk,),
        in_specs=[
            pl.BlockSpec((_C, _COLS_PER_BLOCK), lambda i: (0, i)),
            pl.BlockSpec((_F, _COLS_PER_BLOCK), lambda i: (0, i)),
            pl.BlockSpec((_C, _F), lambda i: (0, 0)),
        ],
        out_specs=pl.BlockSpec((_C, _COLS_PER_BLOCK), lambda i: (0, i)),
        out_shape=jax.ShapeDtypeStruct((_C, _B), jnp.float32),
        scratch_shapes=[pltpu.VMEM((_C, _COLS_PER_BLOCK), jnp.float32)],
    )(xT, attnT, _EXPAND)
    return outT.reshape(_F, _D, _B).transpose(2, 0, 1)


# speedup vs baseline: 1.2049x; 1.2049x over previous
import numpy as np
import jax
import jax.numpy as jnp
from jax import lax
from jax.experimental import pallas as pl
from jax.experimental.pallas import tpu as pltpu

_B, _F, _D = 4096, 100, 64
_C = _F * _D
_COLS_PER_BLOCK = 256

_U32 = np.uint32


def _np_threefry2x32(k1, k2, x0, x1):
    u = lambda v: np.array(v, dtype=_U32)
    rotl = lambda x, d: (x << u(d)) | (x >> u(32 - d))
    ks = [u(k1), u(k2), u(k1) ^ u(k2) ^ u(0x1BD11BDA)]
    rot = [(13, 15, 26, 6), (17, 29, 16, 24)]
    x0, x1 = u(x0) + ks[0], u(x1) + ks[1]
    for i in range(5):
        for r in rot[i % 2]:
            x0 = x0 + x1
            x1 = rotl(x1, r) ^ x0
        x0 = x0 + ks[(i + 1) % 3]
        x1 = x1 + ks[(i + 2) % 3] + u(i + 1)
    return x0, x1


with np.errstate(over="ignore"):
    _KM = _np_threefry2x32(0, 42, 0, 0)
    _KN = _np_threefry2x32(0, 42, 0, 1)

_LO = np.nextafter(np.float32(-1.0), np.float32(0.0), dtype=np.float32)
_HILO = np.float32(np.float32(1.0) - _LO)
_SCALE = np.float32(np.float32(np.sqrt(2.0)) * np.float32(0.05))
_P_CENTER = [np.float32(c) * _SCALE for c in (
    2.81022636e-08, 3.43273939e-07, -3.5233877e-06, -4.39150654e-06,
    0.00021858087, -0.00125372503, -0.00417768164, 0.246640727, 1.50140941)]
_P_TAIL = [np.float32(c) * _SCALE for c in (
    -0.000200214257, 0.000100950558, 0.00134934322, -0.00367342844,
    0.00573950773, -0.0076224613, 0.00943887047, 1.00167406, 2.83297682)]

_EXPAND = np.kron(np.eye(_F, dtype=np.float32),
                  np.ones((_D, 1), dtype=np.float32))


def _tf_rounds(k0, k1, x1):
    ks = [_U32(k0), _U32(k1), _U32(k0) ^ _U32(k1) ^ _U32(0x1BD11BDA)]
    rot = [(13, 15, 26, 6), (17, 29, 16, 24)]
    x0 = jnp.full_like(x1, ks[0])
    x1 = x1 + ks[1]
    for i in range(5):
        for r in rot[i % 2]:
            x0 = x0 + x1
            x1 = ((x1 << _U32(r)) | (x1 >> _U32(32 - r))) ^ x0
        x0 = x0 + ks[(i + 1) % 3]
        x1 = x1 + (ks[(i + 2) % 3] + _U32(i + 1))
    return x0, x1


def _bits(key, flat_idx_u32):
    o0, o1 = _tf_rounds(key[0], key[1], flat_idx_u32)
    return o0 ^ o1


_CHUNK = 1600


def _kern(x_ref, m_ref, e_ref, o_ref):
    cb = x_ref.shape[1]
    col0 = pl.program_id(0) * cb

    f_i = lax.broadcasted_iota(jnp.int32, (_F, cb), 0)
    b_i = lax.broadcasted_iota(jnp.int32, (_F, cb), 1)
    midx = ((col0 + b_i) * _F + f_i).astype(jnp.uint32)
    mbits = _bits(_KM, midx)
    aug = jnp.where((mbits >> _U32(31)) == _U32(0),
                    jnp.float32(1.0), jnp.float32(0.0)) * m_ref[...]

    r_j = lax.broadcasted_iota(jnp.int32, (_CHUNK, cb), 0)
    b_j = lax.broadcasted_iota(jnp.int32, (_CHUNK, cb), 1)
    base = (col0 + b_j) * _C + r_j

    def body(ch, carry):
        del carry
        sl = pl.ds(ch * _CHUNK, _CHUNK)
        nidx = (base + ch * _CHUNK).astype(jnp.uint32)
        nbits = _bits(_KN, nidx)
        fb = (nbits >> _U32(9)) | _U32(0x3F800000)
        f01 = lax.bitcast_convert_type(fb, jnp.float32) - jnp.float32(1.0)
        u = f01 * _HILO + _LO
        w = -jnp.log(jnp.float32(1.0) - u * u)
        tail = w >= jnp.float32(5.0)
        ww = jnp.where(tail, jnp.sqrt(w) - jnp.float32(3.0),
                       w - jnp.float32(2.5))
        p = jnp.where(tail, _P_TAIL[0], _P_CENTER[0])
        for cc, ct in zip(_P_CENTER[1:], _P_TAIL[1:]):
            p = p * ww + jnp.where(tail, ct, cc)
        pert = p * u
        e_ch = lax.dot_general(e_ref[sl, :], aug, (((1,), (0,)), ((), ())),
                               preferred_element_type=jnp.float32)
        o_ref[sl, :] = x_ref[sl, :] + pert * e_ch
        return 0

    lax.fori_loop(0, _C // _CHUNK, body, 0)


def kernel(input_features, attention_mask):
    xT = jnp.transpose(input_features, (1, 2, 0)).reshape(_C, _B)
    attnT = attention_mask.astype(jnp.float32).T
    nblk = _B // _COLS_PER_BLOCK
    outT = pl.pallas_call(
        _kern,
        grid=(nblk,),
        in_specs=[
            pl.BlockSpec((_C, _COLS_PER_BLOCK), lambda i: (0, i)),
            pl.BlockSpec((_F, _COLS_PER_BLOCK), lambda i: (0, i)),
            pl.BlockSpec((_C, _F), lambda i: (0, 0)),
        ],
        out_specs=pl.BlockSpec((_C, _COLS_PER_BLOCK), lambda i: (0, i)),
        out_shape=jax.ShapeDtypeStruct((_C, _B), jnp.float32),
    )(xT, attnT, _EXPAND)
    return outT.reshape(_F, _D, _B).transpose(2, 0, 1)


# speedup vs baseline: 2.1921x; 1.8194x over previous
import numpy as np
import jax
import jax.numpy as jnp
from jax import lax
from jax.experimental import pallas as pl
from jax.experimental.pallas import tpu as pltpu

_B, _F, _D = 4096, 100, 64
_C = _F * _D
_COLS_PER_BLOCK = 256

_U32 = np.uint32


def _np_threefry2x32(k1, k2, x0, x1):
    u = lambda v: np.array(v, dtype=_U32)
    rotl = lambda x, d: (x << u(d)) | (x >> u(32 - d))
    ks = [u(k1), u(k2), u(k1) ^ u(k2) ^ u(0x1BD11BDA)]
    rot = [(13, 15, 26, 6), (17, 29, 16, 24)]
    x0, x1 = u(x0) + ks[0], u(x1) + ks[1]
    for i in range(5):
        for r in rot[i % 2]:
            x0 = x0 + x1
            x1 = rotl(x1, r) ^ x0
        x0 = x0 + ks[(i + 1) % 3]
        x1 = x1 + ks[(i + 2) % 3] + u(i + 1)
    return x0, x1


with np.errstate(over="ignore"):
    _KM = _np_threefry2x32(0, 42, 0, 0)
    _KN = _np_threefry2x32(0, 42, 0, 1)

_LO = np.nextafter(np.float32(-1.0), np.float32(0.0), dtype=np.float32)
_HILO = np.float32(np.float32(1.0) - _LO)
_SCALE = np.float32(np.float32(np.sqrt(2.0)) * np.float32(0.05))
_P_CENTER = [np.float32(c) * _SCALE for c in (
    2.81022636e-08, 3.43273939e-07, -3.5233877e-06, -4.39150654e-06,
    0.00021858087, -0.00125372503, -0.00417768164, 0.246640727, 1.50140941)]
_P_TAIL = [np.float32(c) * _SCALE for c in (
    -0.000200214257, 0.000100950558, 0.00134934322, -0.00367342844,
    0.00573950773, -0.0076224613, 0.00943887047, 1.00167406, 2.83297682)]

_EXPAND = np.kron(np.eye(_F, dtype=np.float32),
                  np.ones((_D, 1), dtype=np.float32))


def _tf_rounds(k0, k1, x1):
    ks = [_U32(k0), _U32(k1), _U32(k0) ^ _U32(k1) ^ _U32(0x1BD11BDA)]
    rot = [(13, 15, 26, 6), (17, 29, 16, 24)]
    x0 = jnp.full_like(x1, ks[0])
    x1 = x1 + ks[1]
    for i in range(5):
        for r in rot[i % 2]:
            x0 = x0 + x1
            x1 = ((x1 << _U32(r)) | (x1 >> _U32(32 - r))) ^ x0
        x0 = x0 + ks[(i + 1) % 3]
        x1 = x1 + (ks[(i + 2) % 3] + _U32(i + 1))
    return x0, x1


def _bits(key, flat_idx_u32):
    o0, o1 = _tf_rounds(key[0], key[1], flat_idx_u32)
    return o0 ^ o1


_CHUNK = 1600


def _kern(x_ref, m_ref, e_ref, o_ref, es_ref):
    cb = x_ref.shape[1]
    col0 = pl.program_id(0) * cb

    f_i = lax.broadcasted_iota(jnp.int32, (_F, cb), 0)
    b_i = lax.broadcasted_iota(jnp.int32, (_F, cb), 1)
    midx = ((col0 + b_i) * _F + f_i).astype(jnp.uint32)
    mbits = _bits(_KM, midx)
    aug = jnp.where((mbits >> _U32(31)) == _U32(0),
                    jnp.float32(1.0), jnp.float32(0.0)) * m_ref[...]
    es_ref[...] = lax.dot_general(e_ref[...], aug, (((1,), (0,)), ((), ())),
                                  preferred_element_type=jnp.float32)

    r_j = lax.broadcasted_iota(jnp.int32, (_CHUNK, cb), 0)
    b_j = lax.broadcasted_iota(jnp.int32, (_CHUNK, cb), 1)
    base = (col0 + b_j) * _C + r_j

    def body(ch, carry):
        del carry
        sl = pl.ds(ch * _CHUNK, _CHUNK)
        nidx = (base + ch * _CHUNK).astype(jnp.uint32)
        nbits = _bits(_KN, nidx)
        fb = (nbits >> _U32(9)) | _U32(0x3F800000)
        f01 = lax.bitcast_convert_type(fb, jnp.float32) - jnp.float32(1.0)
        u = f01 * _HILO + _LO
        w = -jnp.log(jnp.float32(1.0) - u * u)
        tail = w >= jnp.float32(5.0)
        ww = jnp.where(tail, jnp.sqrt(w) - jnp.float32(3.0),
                       w - jnp.float32(2.5))
        p = jnp.where(tail, _P_TAIL[0], _P_CENTER[0])
        for cc, ct in zip(_P_CENTER[1:], _P_TAIL[1:]):
            p = p * ww + jnp.where(tail, ct, cc)
        pert = p * u
        o_ref[sl, :] = x_ref[sl, :] + pert * es_ref[sl, :]
        return 0

    lax.fori_loop(0, _C // _CHUNK, body, 0)


def kernel(input_features, attention_mask):
    xT = jnp.transpose(input_features, (1, 2, 0)).reshape(_C, _B)
    attnT = attention_mask.astype(jnp.float32).T
    nblk = _B // _COLS_PER_BLOCK
    outT = pl.pallas_call(
        _kern,
        grid=(nblk,),
        in_specs=[
            pl.BlockSpec((_C, _COLS_PER_BLOCK), lambda i: (0, i)),
            pl.BlockSpec((_F, _COLS_PER_BLOCK), lambda i: (0, i)),
            pl.BlockSpec((_C, _F), lambda i: (0, 0)),
        ],
        out_specs=pl.BlockSpec((_C, _COLS_PER_BLOCK), lambda i: (0, i)),
        out_shape=jax.ShapeDtypeStruct((_C, _B), jnp.float32),
        scratch_shapes=[pltpu.VMEM((_C, _COLS_PER_BLOCK), jnp.float32)],
    )(xT, attnT, _EXPAND)
    return outT.reshape(_F, _D, _B).transpose(2, 0, 1)
